# baseline (speedup 1.0000x reference)
import functools

import jax
import jax.numpy as jnp
from jax.experimental import pallas as pl
from jax.experimental.pallas import tpu as pltpu


def _inception_kernel(
    x_ref,
    w1_ref, b1_ref,
    ws1_ref, bs1_ref,
    ws2_ref, bs2_ref,
    w1x1_ref, b1x1_ref,
    wa_ref, ba_ref,
    wk_ref, bk_ref,
    wfc_ref, bfc_ref,
    out_ref,
    *, L, Bb,
):
    N = Bb * L

    x = jnp.concatenate([x_ref[b] for b in range(Bb)], axis=1)

    pos = jax.lax.broadcasted_iota(jnp.int32, (1, N), 1) % L
    mask_l = pos >= 1
    mask_r = pos <= L - 2
    kmask = [jnp.logical_and(pos + (k - 3) >= 0, pos + (k - 3) < L)
             for k in range(7)]

    def mm(w, a):
        return jnp.dot(w, a, preferred_element_type=jnp.float32)

    h = mm(w1_ref[...], x) + b1_ref[...]
    sc = mm(ws1_ref[...], h) + bs1_ref[...]

    def inception_block(i, h):
        left = jnp.where(mask_l, pltpu.roll(h, 1, axis=1), h)
        right = jnp.where(mask_r, pltpu.roll(h, N - 1, axis=1), h)
        mp = jnp.maximum(h, jnp.maximum(left, right))
        branch1 = mm(w1x1_ref[i], mp) + b1x1_ref[i]

        b2 = mm(wa_ref[i], h) + ba_ref[i]

        cols = []
        for k in range(7):
            if k == 3:
                cols.append(b2)
            else:
                shifted = pltpu.roll(b2, (3 - k) % N, axis=1)
                cols.append(jnp.where(kmask[k], shifted, 0.0))
        col = jnp.concatenate(cols, axis=0)
        acc = mm(wk_ref[i], col) + bk_ref[i]

        return jnp.concatenate(
            [jnp.maximum(branch1, 0.0), jnp.maximum(acc, 0.0)], axis=0)

    for i in range(3):
        h = inception_block(i, h)
    h = h + sc
    sc2 = jnp.maximum(mm(ws2_ref[...], h) + bs2_ref[...], 0.0)
    for i in range(3, 6):
        h = inception_block(i, h)
    h = h + sc2

    row = jax.lax.broadcasted_iota(jnp.int32, (N, Bb), 0)
    colb = jax.lax.broadcasted_iota(jnp.int32, (N, Bb), 1)
    pool = jnp.where(row // L == colb, 1.0 / L, 0.0).astype(jnp.float32)
    pooled = mm(h, pool)
    y = mm(wfc_ref[...], pooled) + bfc_ref[...]
    out_ref[0] = y


def kernel(x, W1, b1, Ws1, bs1, Ws2, bs2, W1x1, b1x1, Wa, ba, Wk, bk, Wfc, bfc):
    B, k_model, L = x.shape

    Bb = max(1, min(B, 16384 // L))
    B_pad = pl.cdiv(B, Bb) * Bb
    if B_pad != B:
        x = jnp.concatenate(
            [x, jnp.zeros((B_pad - B, k_model, L), x.dtype)], axis=0)
    G = B_pad // Bb
    xp = x.astype(jnp.float32)

    args = [W1, b1, Ws1, bs1, Ws2, bs2, W1x1, b1x1, Wa, ba, Wk, bk,
            Wfc, bfc.reshape(1, 1)]

    def full_spec(a):
        nd = a.ndim
        return pl.BlockSpec(a.shape, lambda g, nd=nd: (0,) * nd)

    out = pl.pallas_call(
        functools.partial(_inception_kernel, L=L, Bb=Bb),
        out_shape=jax.ShapeDtypeStruct((G, 1, Bb), jnp.float32),
        grid=(G,),
        in_specs=[pl.BlockSpec((Bb, k_model, L), lambda g: (g, 0, 0))]
                 + [full_spec(a) for a in args],
        out_specs=pl.BlockSpec((1, 1, Bb), lambda g: (g, 0, 0)),
        compiler_params=pltpu.CompilerParams(
            dimension_semantics=("parallel",)),
    )(xp, *args)
    return out.reshape(B_pad, 1)[:B]


# speedup vs baseline: 1.4829x; 1.4829x over previous
import functools

import jax
import jax.numpy as jnp
from jax.experimental import pallas as pl
from jax.experimental.pallas import tpu as pltpu

try:
    from jax.experimental.shard_map import shard_map as _shard_map
except ImportError:
    from jax import shard_map as _shard_map
from jax.sharding import Mesh, PartitionSpec as P


def _inception_kernel(
    x_ref,
    w1_ref, b1_ref,
    ws1_ref, bs1_ref,
    ws2_ref, bs2_ref,
    w1x1_ref, b1x1_ref,
    wa_ref, ba_ref,
    wk_ref, bk_ref,
    wfc_ref, bfc_ref,
    out_ref,
    *, L, Bb,
):
    N = Bb * L

    x = jnp.concatenate([x_ref[b] for b in range(Bb)], axis=1)

    pos = jax.lax.broadcasted_iota(jnp.int32, (1, N), 1) % L
    mask_l = pos >= 1
    mask_r = pos <= L - 2
    kmask = [jnp.logical_and(pos + (k - 3) >= 0, pos + (k - 3) < L)
             for k in range(7)]

    def mm(w, a):
        return jnp.dot(w, a, preferred_element_type=jnp.float32)

    h = mm(w1_ref[...], x) + b1_ref[...]
    sc = mm(ws1_ref[...], h) + bs1_ref[...]

    def inception_block(i, h):
        left = jnp.where(mask_l, pltpu.roll(h, 1, axis=1), h)
        right = jnp.where(mask_r, pltpu.roll(h, N - 1, axis=1), h)
        mp = jnp.maximum(h, jnp.maximum(left, right))
        branch1 = mm(w1x1_ref[i], mp) + b1x1_ref[i]

        b2 = mm(wa_ref[i], h) + ba_ref[i]

        cols = []
        for k in range(7):
            if k == 3:
                cols.append(b2)
            else:
                shifted = pltpu.roll(b2, (3 - k) % N, axis=1)
                cols.append(jnp.where(kmask[k], shifted, 0.0))
        col = jnp.concatenate(cols, axis=0)
        acc = mm(wk_ref[i], col) + bk_ref[i]

        return jnp.concatenate(
            [jnp.maximum(branch1, 0.0), jnp.maximum(acc, 0.0)], axis=0)

    for i in range(3):
        h = inception_block(i, h)
    h = h + sc
    sc2 = jnp.maximum(mm(ws2_ref[...], h) + bs2_ref[...], 0.0)
    for i in range(3, 6):
        h = inception_block(i, h)
    h = h + sc2

    row = jax.lax.broadcasted_iota(jnp.int32, (N, Bb), 0)
    colb = jax.lax.broadcasted_iota(jnp.int32, (N, Bb), 1)
    pool = jnp.where(row // L == colb, 1.0 / L, 0.0).astype(jnp.float32)
    pooled = mm(h, pool)
    y = mm(wfc_ref[...], pooled) + bfc_ref[...]
    out_ref[0] = y


def kernel(x, W1, b1, Ws1, bs1, Ws2, bs2, W1x1, b1x1, Wa, ba, Wk, bk, Wfc, bfc):
    B, k_model, L = x.shape

    Bb = max(1, min(B, 8192 // L))
    B_pad = pl.cdiv(B, Bb) * Bb
    if B_pad != B:
        x = jnp.concatenate(
            [x, jnp.zeros((B_pad - B, k_model, L), x.dtype)], axis=0)
    G = B_pad // Bb
    xp = x.astype(jnp.float32)

    args = [W1, b1, Ws1, bs1, Ws2, bs2, W1x1, b1x1, Wa, ba, Wk, bk,
            Wfc, bfc.reshape(1, 1)]

    def full_spec(a):
        nd = a.ndim
        return pl.BlockSpec(a.shape, lambda g, nd=nd: (0,) * nd)

    def run(xs, *ws):
        g_loc = xs.shape[0] // Bb
        return pl.pallas_call(
            functools.partial(_inception_kernel, L=L, Bb=Bb),
            out_shape=jax.ShapeDtypeStruct((g_loc, 1, Bb), jnp.float32),
            grid=(g_loc,),
            in_specs=[pl.BlockSpec((Bb, k_model, L), lambda g: (g, 0, 0))]
                     + [full_spec(a) for a in ws],
            out_specs=pl.BlockSpec((1, 1, Bb), lambda g: (g, 0, 0)),
            compiler_params=pltpu.CompilerParams(
                dimension_semantics=("parallel",)),
        )(xs, *ws)

    devs = jax.devices()
    if len(devs) >= 2 and G % 2 == 0:
        mesh = Mesh(devs[:2], ("b",))
        out = _shard_map(
            run, mesh=mesh,
            in_specs=(P("b"),) + (P(),) * len(args),
            out_specs=P("b"),
            check_rep=False,
        )(xp, *args)
    else:
        out = run(xp, *args)
    return out.reshape(B_pad, 1)[:B]


# speedup vs baseline: 1.5051x; 1.0150x over previous
import functools

import jax
import jax.numpy as jnp
from jax.experimental import pallas as pl
from jax.experimental.pallas import tpu as pltpu

try:
    from jax.experimental.shard_map import shard_map as _shard_map
except ImportError:
    from jax import shard_map as _shard_map
from jax.sharding import Mesh, PartitionSpec as P


def _inception_kernel(
    x_ref,
    w1_ref, b1_ref,
    ws1_ref, bs1_ref,
    ws2_ref, bs2_ref,
    w1x1_ref, b1x1_ref,
    wa_ref, ba_ref,
    wk_ref, bk_ref,
    wfc_ref, bfc_ref,
    out_ref,
    *, L, Bb,
):
    N = Bb * L

    x = jnp.concatenate([x_ref[b] for b in range(Bb)], axis=1)

    pos = jax.lax.broadcasted_iota(jnp.int32, (1, N), 1) % L
    mask_l = pos >= 1
    mask_r = pos <= L - 2
    kmask = [jnp.logical_and(pos + (k - 3) >= 0, pos + (k - 3) < L)
             for k in range(7)]

    def mm(w, a):
        return jnp.dot(w, a, preferred_element_type=jnp.float32)

    h = mm(w1_ref[...], x) + b1_ref[...]
    sc = mm(ws1_ref[...], h) + bs1_ref[...]

    def inception_block(i, h):
        left = jnp.where(mask_l, pltpu.roll(h, 1, axis=1), h)
        right = jnp.where(mask_r, pltpu.roll(h, N - 1, axis=1), h)
        mp = jnp.maximum(h, jnp.maximum(left, right))
        branch1 = mm(w1x1_ref[i], mp) + b1x1_ref[i]

        b2 = mm(wa_ref[i], h) + ba_ref[i]

        cols = []
        for k in range(7):
            if k == 3:
                cols.append(b2)
            else:
                shifted = pltpu.roll(b2, (3 - k) % N, axis=1)
                cols.append(jnp.where(kmask[k], shifted, 0.0))
        col = jnp.concatenate(cols, axis=0)
        acc = mm(wk_ref[i], col) + bk_ref[i]

        return jnp.concatenate(
            [jnp.maximum(branch1, 0.0), jnp.maximum(acc, 0.0)], axis=0)

    for i in range(3):
        h = inception_block(i, h)
    h = h + sc
    sc2 = jnp.maximum(mm(ws2_ref[...], h) + bs2_ref[...], 0.0)
    for i in range(3, 6):
        h = inception_block(i, h)
    h = h + sc2

    row = jax.lax.broadcasted_iota(jnp.int32, (N, Bb), 0)
    colb = jax.lax.broadcasted_iota(jnp.int32, (N, Bb), 1)
    pool = jnp.where(row // L == colb, 1.0 / L, 0.0).astype(jnp.float32)
    pooled = mm(h, pool)
    y = mm(wfc_ref[...], pooled) + bfc_ref[...]
    out_ref[0] = y


def kernel(x, W1, b1, Ws1, bs1, Ws2, bs2, W1x1, b1x1, Wa, ba, Wk, bk, Wfc, bfc):
    B, k_model, L = x.shape

    Bb = max(1, min(B, 8192 // L))
    B_pad = pl.cdiv(B, Bb) * Bb
    if B_pad != B:
        x = jnp.concatenate(
            [x, jnp.zeros((B_pad - B, k_model, L), x.dtype)], axis=0)
    G = B_pad // Bb
    xp = x.astype(jnp.float32)

    args = [W1, b1, Ws1, bs1, Ws2, bs2, W1x1, b1x1, Wa, ba, Wk, bk,
            Wfc, bfc.reshape(1, 1)]

    def full_spec(a):
        nd = a.ndim
        return pl.BlockSpec(a.shape, lambda g, nd=nd: (0,) * nd)

    def run(xs, *ws):
        g_loc = xs.shape[0] // Bb
        out = pl.pallas_call(
            functools.partial(_inception_kernel, L=L, Bb=Bb),
            out_shape=jax.ShapeDtypeStruct((g_loc, 1, Bb), jnp.float32),
            grid=(g_loc,),
            in_specs=[pl.BlockSpec((Bb, k_model, L), lambda g: (g, 0, 0))]
                     + [full_spec(a) for a in ws],
            out_specs=pl.BlockSpec((1, 1, Bb), lambda g: (g, 0, 0)),
            compiler_params=pltpu.CompilerParams(
                dimension_semantics=("parallel",)),
        )(xs, *ws)
        return out.reshape(g_loc * Bb, 1)

    devs = jax.devices()
    if len(devs) >= 2 and G % 2 == 0:
        mesh = Mesh(devs[:2], ("b",))
        out = _shard_map(
            run, mesh=mesh,
            in_specs=(P("b"),) + (P(),) * len(args),
            out_specs=P("b"),
            check_rep=False,
        )(xp, *args)
    else:
        out = run(xp, *args)
    return out[:B]
